# baseline (device time: 23241 ns/iter reference)
import jax
import jax.numpy as jnp
from jax import lax
from jax.experimental import pallas as pl
from jax.experimental.pallas import tpu as pltpu


def kernel(x, router, W1, W2):
    t, d = x.shape
    e_loc, _, f = W1.shape
    n_chunks = 2
    h = t // n_chunks
    f32 = jnp.float32
    bf16 = jnp.bfloat16

    def body(x_ref, r_ref, w1_hbm, w2_hbm, out_ref,
             w1_vmem, w2_vmem, x_send, xr_comm, r_send, r_comm,
             wt_send, wt_comm, part_send, part_comm,
             w_sems, send_sems, recv_sems):
        my_x = lax.axis_index("x")
        my_y = lax.axis_index("y")
        my_z = lax.axis_index("z")
        peer = (my_x, 1 - my_y, my_z)
        i_am_lo = my_y == 0

        cp1 = pltpu.make_async_copy(w1_hbm, w1_vmem, w_sems.at[0])
        cp2 = pltpu.make_async_copy(w2_hbm, w2_vmem, w_sems.at[1])
        cp1.start()
        cp2.start()

        barrier = pltpu.get_barrier_semaphore()
        pl.semaphore_signal(barrier, inc=1)
        pl.semaphore_wait(barrier, 1)

        x_send[...] = x_ref[...].astype(bf16)
        rdma_x = pltpu.make_async_remote_copy(
            src_ref=x_send, dst_ref=xr_comm,
            send_sem=send_sems.at[0], recv_sem=recv_sems.at[0],
            device_id=peer, device_id_type=pl.DeviceIdType.MESH)
        rdma_x.start()
        r_send[...] = jnp.transpose(r_ref[...])
        rdma_r = pltpu.make_async_remote_copy(
            src_ref=r_send, dst_ref=r_comm,
            send_sem=send_sems.at[1], recv_sem=recv_sems.at[1],
            device_id=peer, device_id_type=pl.DeviceIdType.MESH)
        rdma_r.start()
        g_mine = jnp.dot(x_ref[...], r_ref[...],
                         precision=lax.Precision.HIGHEST)
        rdma_r.wait()

        g_peer = lax.dot_general(
            x_ref[...], r_comm[...], (((1,), (1,)), ((), ())),
            precision=lax.Precision.HIGHEST)
        g = jnp.where(i_am_lo,
                      jnp.concatenate([g_mine, g_peer], axis=1),
                      jnp.concatenate([g_peer, g_mine], axis=1))

        m1 = jnp.max(g, axis=1, keepdims=True)
        is_top1 = g == m1
        g_rest = jnp.where(is_top1, -jnp.inf, g)
        m2 = jnp.max(g_rest, axis=1, keepdims=True)
        sel = is_top1 | (g_rest == m2)
        ex = jnp.where(sel, jnp.exp(g - m1), 0.0)
        w = ex / jnp.sum(ex, axis=1, keepdims=True)

        w_mine = jnp.where(i_am_lo, w[:, :e_loc], w[:, e_loc:])
        wt_send[...] = jnp.transpose(
            jnp.where(i_am_lo, w[:, e_loc:], w[:, :e_loc]))
        rdma_w = pltpu.make_async_remote_copy(
            src_ref=wt_send, dst_ref=wt_comm,
            send_sem=send_sems.at[2], recv_sem=recv_sems.at[2],
            device_id=peer, device_id_type=pl.DeviceIdType.MESH)
        rdma_w.start()

        cp1.wait()
        cp2.wait()
        w1b = [w1_vmem[e].astype(bf16) for e in range(e_loc)]
        w2b = [w2_vmem[e].astype(bf16) for e in range(e_loc)]
        rdma_x.wait()
        rdma_w.wait()

        def ffn(xv, wcols):
            n = xv.shape[0]
            acc = jnp.zeros((n, d), f32)
            for e in range(e_loc):
                hh = jnp.maximum(
                    jnp.dot(xv, w1b[e], preferred_element_type=f32), 0)
                acc += jnp.dot(hh.astype(bf16), w2b[e],
                               preferred_element_type=f32) * wcols[e]
            return acc

        rdma_p = []
        for c in range(n_chunks):
            sl = pl.ds(c * h, h)
            wcols = [jnp.transpose(wt_comm[e:e + 1, sl])
                     for e in range(e_loc)]
            part_send[sl, :] = ffn(xr_comm[sl, :], wcols).astype(bf16)
            rp = pltpu.make_async_remote_copy(
                src_ref=part_send.at[sl, :], dst_ref=part_comm.at[sl, :],
                send_sem=send_sems.at[3 + c], recv_sem=recv_sems.at[3 + c],
                device_id=peer, device_id_type=pl.DeviceIdType.MESH)
            rp.start()
            rdma_p.append(rp)

        acc_mine = ffn(x_send[...],
                       [w_mine[:, e:e + 1] for e in range(e_loc)])
        for c, rp in enumerate(rdma_p):
            sl = pl.ds(c * h, h)
            rp.wait()
            out_ref[sl, :] = acc_mine[c * h:(c + 1) * h, :] \
                + part_comm[sl, :].astype(f32)

    return pl.pallas_call(
        body,
        out_shape=jax.ShapeDtypeStruct((t, d), f32),
        in_specs=[
            pl.BlockSpec(memory_space=pltpu.VMEM),
            pl.BlockSpec(memory_space=pltpu.VMEM),
            pl.BlockSpec(memory_space=pltpu.MemorySpace.HBM),
            pl.BlockSpec(memory_space=pltpu.MemorySpace.HBM),
        ],
        out_specs=pl.BlockSpec(memory_space=pltpu.VMEM),
        scratch_shapes=[
            pltpu.VMEM((e_loc, d, f), f32),
            pltpu.VMEM((e_loc, f, d), f32),
            pltpu.VMEM((t, d), bf16),
            pltpu.VMEM((t, d), bf16),
            pltpu.VMEM((e_loc, d), f32),
            pltpu.VMEM((e_loc, d), f32),
            pltpu.VMEM((e_loc, t), f32),
            pltpu.VMEM((e_loc, t), f32),
            pltpu.VMEM((t, d), bf16),
            pltpu.VMEM((t, d), bf16),
            pltpu.SemaphoreType.DMA((2,)),
            pltpu.SemaphoreType.DMA((3 + n_chunks,)),
            pltpu.SemaphoreType.DMA((3 + n_chunks,)),
        ],
        compiler_params=pltpu.CompilerParams(collective_id=0),
    )(x, router, W1, W2)


# device time: 23174 ns/iter; 1.0029x vs baseline; 1.0029x over previous
import jax
import jax.numpy as jnp
from jax import lax
from jax.experimental import pallas as pl
from jax.experimental.pallas import tpu as pltpu


def kernel(x, router, W1, W2):
    t, d = x.shape
    e_loc, _, f = W1.shape
    n_chunks = 2
    h = t // n_chunks
    f32 = jnp.float32
    bf16 = jnp.bfloat16

    def body(x_ref, r_ref, w1_hbm, w2_hbm, out_ref,
             w1_vmem, w2_vmem, x_send, xr_comm, r_send, r_comm,
             wt_send, wt_comm, part_send, part_comm,
             w_sems, send_sems, recv_sems):
        my_x = lax.axis_index("x")
        my_y = lax.axis_index("y")
        my_z = lax.axis_index("z")
        peer = (my_x, 1 - my_y, my_z)
        i_am_lo = my_y == 0

        cp1 = pltpu.make_async_copy(w1_hbm, w1_vmem, w_sems.at[0])
        cp2 = pltpu.make_async_copy(w2_hbm, w2_vmem, w_sems.at[1])
        cp1.start()
        cp2.start()

        barrier = pltpu.get_barrier_semaphore()
        pl.semaphore_signal(barrier, inc=1, device_id=peer,
                            device_id_type=pl.DeviceIdType.MESH)
        pl.semaphore_wait(barrier, 1)

        x_send[...] = x_ref[...].astype(bf16)
        rdma_x = []
        for c in range(n_chunks):
            sl = pl.ds(c * h, h)
            rx = pltpu.make_async_remote_copy(
                src_ref=x_send.at[sl, :], dst_ref=xr_comm.at[sl, :],
                send_sem=send_sems.at[c], recv_sem=recv_sems.at[c],
                device_id=peer, device_id_type=pl.DeviceIdType.MESH)
            rx.start()
            rdma_x.append(rx)
        r_send[...] = jnp.transpose(r_ref[...])
        rdma_r = pltpu.make_async_remote_copy(
            src_ref=r_send, dst_ref=r_comm,
            send_sem=send_sems.at[n_chunks], recv_sem=recv_sems.at[n_chunks],
            device_id=peer, device_id_type=pl.DeviceIdType.MESH)
        rdma_r.start()
        g_mine = jnp.dot(x_ref[...], r_ref[...],
                         precision=lax.Precision.HIGHEST)
        rdma_r.wait()

        g_peer = lax.dot_general(
            x_ref[...], r_comm[...], (((1,), (1,)), ((), ())),
            precision=lax.Precision.HIGHEST)
        g = jnp.where(i_am_lo,
                      jnp.concatenate([g_mine, g_peer], axis=1),
                      jnp.concatenate([g_peer, g_mine], axis=1))

        m1 = jnp.max(g, axis=1, keepdims=True)
        is_top1 = g == m1
        g_rest = jnp.where(is_top1, -jnp.inf, g)
        m2 = jnp.max(g_rest, axis=1, keepdims=True)
        sel = is_top1 | (g_rest == m2)
        ex = jnp.where(sel, jnp.exp(g - m1), 0.0)
        w = ex / jnp.sum(ex, axis=1, keepdims=True)

        w_mine = jnp.where(i_am_lo, w[:, :e_loc], w[:, e_loc:])
        wt_send[...] = jnp.transpose(
            jnp.where(i_am_lo, w[:, e_loc:], w[:, :e_loc]))
        rdma_w = pltpu.make_async_remote_copy(
            src_ref=wt_send, dst_ref=wt_comm,
            send_sem=send_sems.at[n_chunks + 1],
            recv_sem=recv_sems.at[n_chunks + 1],
            device_id=peer, device_id_type=pl.DeviceIdType.MESH)
        rdma_w.start()

        cp1.wait()
        cp2.wait()
        w1b = [w1_vmem[e].astype(bf16) for e in range(e_loc)]
        w2b = [w2_vmem[e].astype(bf16) for e in range(e_loc)]
        rdma_w.wait()

        def ffn(xv, wcols):
            n = xv.shape[0]
            acc = jnp.zeros((n, d), f32)
            for e in range(e_loc):
                hh = jnp.maximum(
                    jnp.dot(xv, w1b[e], preferred_element_type=f32), 0)
                acc += jnp.dot(hh.astype(bf16), w2b[e],
                               preferred_element_type=f32) * wcols[e]
            return acc

        rdma_p = []
        for c in range(n_chunks):
            sl = pl.ds(c * h, h)
            rdma_x[c].wait()
            wcols = [jnp.transpose(wt_comm[e:e + 1, sl])
                     for e in range(e_loc)]
            part_send[sl, :] = ffn(xr_comm[sl, :], wcols).astype(bf16)
            rp = pltpu.make_async_remote_copy(
                src_ref=part_send.at[sl, :], dst_ref=part_comm.at[sl, :],
                send_sem=send_sems.at[n_chunks + 2 + c],
                recv_sem=recv_sems.at[n_chunks + 2 + c],
                device_id=peer, device_id_type=pl.DeviceIdType.MESH)
            rp.start()
            rdma_p.append(rp)

        acc_mine = ffn(x_send[...],
                       [w_mine[:, e:e + 1] for e in range(e_loc)])
        for c, rp in enumerate(rdma_p):
            sl = pl.ds(c * h, h)
            rp.wait()
            out_ref[sl, :] = acc_mine[c * h:(c + 1) * h, :] \
                + part_comm[sl, :].astype(f32)

    return pl.pallas_call(
        body,
        out_shape=jax.ShapeDtypeStruct((t, d), f32),
        in_specs=[
            pl.BlockSpec(memory_space=pltpu.VMEM),
            pl.BlockSpec(memory_space=pltpu.VMEM),
            pl.BlockSpec(memory_space=pltpu.MemorySpace.HBM),
            pl.BlockSpec(memory_space=pltpu.MemorySpace.HBM),
        ],
        out_specs=pl.BlockSpec(memory_space=pltpu.VMEM),
        scratch_shapes=[
            pltpu.VMEM((e_loc, d, f), f32),
            pltpu.VMEM((e_loc, f, d), f32),
            pltpu.VMEM((t, d), bf16),
            pltpu.VMEM((t, d), bf16),
            pltpu.VMEM((e_loc, d), f32),
            pltpu.VMEM((e_loc, d), f32),
            pltpu.VMEM((e_loc, t), f32),
            pltpu.VMEM((e_loc, t), f32),
            pltpu.VMEM((t, d), bf16),
            pltpu.VMEM((t, d), bf16),
            pltpu.SemaphoreType.DMA((2,)),
            pltpu.SemaphoreType.DMA((2 + 2 * n_chunks,)),
            pltpu.SemaphoreType.DMA((2 + 2 * n_chunks,)),
        ],
        compiler_params=pltpu.CompilerParams(collective_id=0),
    )(x, router, W1, W2)
